# Initial kernel scaffold; baseline (speedup 1.0000x reference)
import jax
import jax.numpy as jnp
from jax.experimental import pallas as pl


def kernel(feat, edge_index):
    raise NotImplementedError("write your pallas kernel here")


# speedup vs baseline: 3.5033x; 3.5033x over previous
import functools

import jax
import jax.numpy as jnp
from jax import lax
from jax.experimental import pallas as pl
from jax.experimental.pallas import tpu as pltpu
from jax.experimental.pallas import tpu_sc as plsc

N_NODES = 10000
N_EDGES = 320000
D_FEAT = 128

_NC = 2
_NS = 16
_NW = _NC * _NS
_EPW = N_EDGES // _NW
_C = 80
_NCHUNKS = _EPW // _C


def _sc_body(feat_hbm, src_hbm, dst_hbm, out_hbm,
             idx_u, idx_v, rows_u, rows_v, sem_u, sem_v):
    wid = lax.axis_index("s") * _NC + lax.axis_index("c")
    tile_base = wid * _EPW

    def chunk_body(ci, carry):
        base = tile_base + ci * _C
        pltpu.sync_copy(src_hbm.at[pl.ds(base, _C)], idx_u)
        pltpu.sync_copy(dst_hbm.at[pl.ds(base, _C)], idx_v)
        cu = pltpu.async_copy(feat_hbm.at[idx_u], rows_u, sem_u)
        cv = pltpu.async_copy(feat_hbm.at[idx_v], rows_v, sem_v)
        cu.wait()
        cv.wait()

        def mul_body(e, c2):
            for j in range(D_FEAT // 16):
                s = pl.ds(j * 16, 16)
                rows_u[e, s] = rows_u[e, s] * rows_v[e, s]
            return c2

        lax.fori_loop(0, _C, mul_body, 0)
        pltpu.sync_copy(rows_u, out_hbm.at[pl.ds(base, _C)])
        return carry

    lax.fori_loop(0, _NCHUNKS, chunk_body, 0)


@functools.partial(jax.jit, static_argnames=())
def _gather_mul(feat, src, dst):
    mesh = plsc.VectorSubcoreMesh(core_axis_name="c", subcore_axis_name="s")
    f = pl.kernel(
        _sc_body,
        mesh=mesh,
        out_type=jax.ShapeDtypeStruct((N_EDGES, D_FEAT), jnp.float32),
        scratch_types=[
            pltpu.VMEM((_C,), jnp.int32),
            pltpu.VMEM((_C,), jnp.int32),
            pltpu.VMEM((_C, D_FEAT), jnp.float32),
            pltpu.VMEM((_C, D_FEAT), jnp.float32),
            pltpu.SemaphoreType.DMA,
            pltpu.SemaphoreType.DMA,
        ],
    )
    return f(feat, src, dst)


def kernel(feat, edge_index):
    src = edge_index[0].astype(jnp.int32)
    dst = edge_index[1].astype(jnp.int32)
    return _gather_mul(feat, src, dst)


# speedup vs baseline: 7.6328x; 2.1788x over previous
import functools

import jax
import jax.numpy as jnp
from jax import lax
from jax.experimental import pallas as pl
from jax.experimental.pallas import tpu as pltpu
from jax.experimental.pallas import tpu_sc as plsc

N_NODES = 10000
N_EDGES = 320000
D_FEAT = 128

_NC = 2
_NS = 16
_NW = _NC * _NS
_EPW = N_EDGES // _NW
_C = 80
_NCHUNKS = _EPW // _C
_NBUF = 2
_NMAIN = _NCHUNKS - (_NCHUNKS % _NBUF)


def _sc_body(feat_hbm, src_hbm, dst_hbm, out_hbm,
             idx_u, idx_v,
             ru0, rv0, ou0, ru1, rv1, ou1,
             sgu0, sgv0, swb0, sgu1, sgv1, swb1):
    wid = lax.axis_index("s") * _NC + lax.axis_index("c")
    tile_base = wid * _EPW
    pltpu.sync_copy(src_hbm.at[pl.ds(tile_base, _EPW)], idx_u)
    pltpu.sync_copy(dst_hbm.at[pl.ds(tile_base, _EPW)], idx_v)

    rows_u = (ru0, ru1)
    rows_v = (rv0, rv1)
    out_b = (ou0, ou1)
    sem_gu = (sgu0, sgu1)
    sem_gv = (sgv0, sgv1)
    sem_wb = (swb0, swb1)

    def issue_gather(c, b):
        off = c * _C
        pltpu.async_copy(feat_hbm.at[idx_u.at[pl.ds(off, _C)]], rows_u[b], sem_gu[b])
        pltpu.async_copy(feat_hbm.at[idx_v.at[pl.ds(off, _C)]], rows_v[b], sem_gv[b])

    def wait_gather(b):
        pltpu.make_async_copy(feat_hbm.at[pl.ds(0, _C)], rows_u[b], sem_gu[b]).wait()
        pltpu.make_async_copy(feat_hbm.at[pl.ds(0, _C)], rows_v[b], sem_gv[b]).wait()

    def issue_wb(c, b):
        base = tile_base + c * _C
        pltpu.async_copy(out_b[b], out_hbm.at[pl.ds(base, _C)], sem_wb[b])

    def wait_wb(b):
        pltpu.make_async_copy(out_b[b], out_hbm.at[pl.ds(0, _C)], sem_wb[b]).wait()

    def compute(b):
        ru, rv, ob = rows_u[b], rows_v[b], out_b[b]

        def mul_body(e, c2):
            for j in range(D_FEAT // 16):
                s = pl.ds(j * 16, 16)
                ob[e, s] = ru[e, s] * rv[e, s]
            return c2

        lax.fori_loop(0, _C, mul_body, 0)

    issue_gather(0, 0)
    issue_gather(1, 1)

    def outer(i, carry):
        for b in range(_NBUF):
            c = i * _NBUF + b
            wait_gather(b)

            @pl.when(i >= 1)
            def _():
                wait_wb(b)

            compute(b)
            issue_wb(c, b)
            if b == 0:
                issue_gather(c + _NBUF, b)
            else:
                @pl.when(c + _NBUF < _NCHUNKS)
                def _():
                    issue_gather(c + _NBUF, b)
        return carry

    lax.fori_loop(0, _NMAIN // _NBUF, outer, 0)

    for c in range(_NMAIN, _NCHUNKS):
        b = c % _NBUF
        wait_gather(b)
        wait_wb(b)
        compute(b)
        issue_wb(c, b)
    for b in range(_NBUF):
        wait_wb(b)


@jax.jit
def _gather_mul(feat, src, dst):
    mesh = plsc.VectorSubcoreMesh(core_axis_name="c", subcore_axis_name="s")
    f = pl.kernel(
        _sc_body,
        mesh=mesh,
        out_type=jax.ShapeDtypeStruct((N_EDGES, D_FEAT), jnp.float32),
        scratch_types=[
            pltpu.VMEM((_EPW,), jnp.int32),
            pltpu.VMEM((_EPW,), jnp.int32),
            pltpu.VMEM((_C, D_FEAT), jnp.float32),
            pltpu.VMEM((_C, D_FEAT), jnp.float32),
            pltpu.VMEM((_C, D_FEAT), jnp.float32),
            pltpu.VMEM((_C, D_FEAT), jnp.float32),
            pltpu.VMEM((_C, D_FEAT), jnp.float32),
            pltpu.VMEM((_C, D_FEAT), jnp.float32),
            pltpu.SemaphoreType.DMA,
            pltpu.SemaphoreType.DMA,
            pltpu.SemaphoreType.DMA,
            pltpu.SemaphoreType.DMA,
            pltpu.SemaphoreType.DMA,
            pltpu.SemaphoreType.DMA,
        ],
    )
    return f(feat, src, dst)


def kernel(feat, edge_index):
    src = edge_index[0].astype(jnp.int32)
    dst = edge_index[1].astype(jnp.int32)
    return _gather_mul(feat, src, dst)


# speedup vs baseline: 9.1822x; 1.2030x over previous
import functools

import jax
import jax.numpy as jnp
from jax import lax
from jax.experimental import pallas as pl
from jax.experimental.pallas import tpu as pltpu
from jax.experimental.pallas import tpu_sc as plsc

N_NODES = 10000
N_EDGES = 320000
D_FEAT = 128

_NC = 2
_NS = 16
_NW = _NC * _NS
_EPW = N_EDGES // _NW
_C = 40
_NCHUNKS = _EPW // _C
_NBUF = 2
_NMAIN = _NCHUNKS - (_NCHUNKS % _NBUF)


_ROWS_PER_TILE = 624
_ROWS_TAIL = N_NODES - _NS * _ROWS_PER_TILE


def _sc_body(feat_hbm, src_hbm, dst_hbm, out_hbm,
             feat_sp, idx_u, idx_v,
             ru0, rv0, ou0, ru1, rv1, ou1,
             sgu0, sgv0, swb0, sgu1, sgv1, swb1):
    sid = lax.axis_index("s")
    wid = sid * _NC + lax.axis_index("c")
    tile_base = wid * _EPW
    pltpu.async_copy(feat_hbm.at[pl.ds(sid * _ROWS_PER_TILE, _ROWS_PER_TILE)],
                     feat_sp.at[pl.ds(sid * _ROWS_PER_TILE, _ROWS_PER_TILE)],
                     sgu0).wait()

    @pl.when(sid == 0)
    def _():
        tail = _NS * _ROWS_PER_TILE
        pltpu.async_copy(feat_hbm.at[pl.ds(tail, _ROWS_TAIL)],
                         feat_sp.at[pl.ds(tail, _ROWS_TAIL)],
                         sgu0).wait()
    pltpu.sync_copy(src_hbm.at[pl.ds(tile_base, _EPW)], idx_u)
    pltpu.sync_copy(dst_hbm.at[pl.ds(tile_base, _EPW)], idx_v)
    plsc.subcore_barrier()

    rows_u = (ru0, ru1)
    rows_v = (rv0, rv1)
    out_b = (ou0, ou1)
    sem_gu = (sgu0, sgu1)
    sem_gv = (sgv0, sgv1)
    sem_wb = (swb0, swb1)

    def issue_gather(c, b):
        off = c * _C
        pltpu.async_copy(feat_sp.at[idx_u.at[pl.ds(off, _C)]], rows_u[b], sem_gu[b])
        pltpu.async_copy(feat_sp.at[idx_v.at[pl.ds(off, _C)]], rows_v[b], sem_gv[b])

    def wait_gather(b):
        pltpu.make_async_copy(feat_sp.at[pl.ds(0, _C)], rows_u[b], sem_gu[b]).wait()
        pltpu.make_async_copy(feat_sp.at[pl.ds(0, _C)], rows_v[b], sem_gv[b]).wait()

    def issue_wb(c, b):
        base = tile_base + c * _C
        pltpu.async_copy(out_b[b], out_hbm.at[pl.ds(base, _C)], sem_wb[b])

    def wait_wb(b):
        pltpu.make_async_copy(out_b[b], out_hbm.at[pl.ds(0, _C)], sem_wb[b]).wait()

    def compute(b):
        ru, rv, ob = rows_u[b], rows_v[b], out_b[b]

        @plsc.parallel_loop(0, _C, 1, unroll=4)
        def _(e):
            for j in range(D_FEAT // 16):
                s = pl.ds(j * 16, 16)
                ob[e, s] = ru[e, s] * rv[e, s]

    issue_gather(0, 0)
    issue_gather(1, 1)

    def outer(i, carry):
        for b in range(_NBUF):
            c = i * _NBUF + b
            wait_gather(b)

            @pl.when(i >= 1)
            def _():
                wait_wb(b)

            compute(b)
            issue_wb(c, b)

            @pl.when(c + _NBUF < _NCHUNKS)
            def _():
                issue_gather(c + _NBUF, b)
        return carry

    lax.fori_loop(0, _NMAIN // _NBUF, outer, 0)

    for c in range(_NMAIN, _NCHUNKS):
        b = c % _NBUF
        wait_gather(b)
        wait_wb(b)
        compute(b)
        issue_wb(c, b)
    for b in range(_NBUF):
        wait_wb(b)


@jax.jit
def _gather_mul(feat, src, dst):
    mesh = plsc.VectorSubcoreMesh(core_axis_name="c", subcore_axis_name="s")
    f = pl.kernel(
        _sc_body,
        mesh=mesh,
        out_type=jax.ShapeDtypeStruct((N_EDGES, D_FEAT), jnp.float32),
        scratch_types=[
            pltpu.VMEM_SHARED((N_NODES, D_FEAT), jnp.float32),
            pltpu.VMEM((_EPW,), jnp.int32),
            pltpu.VMEM((_EPW,), jnp.int32),
            pltpu.VMEM((_C, D_FEAT), jnp.float32),
            pltpu.VMEM((_C, D_FEAT), jnp.float32),
            pltpu.VMEM((_C, D_FEAT), jnp.float32),
            pltpu.VMEM((_C, D_FEAT), jnp.float32),
            pltpu.VMEM((_C, D_FEAT), jnp.float32),
            pltpu.VMEM((_C, D_FEAT), jnp.float32),
            pltpu.SemaphoreType.DMA,
            pltpu.SemaphoreType.DMA,
            pltpu.SemaphoreType.DMA,
            pltpu.SemaphoreType.DMA,
            pltpu.SemaphoreType.DMA,
            pltpu.SemaphoreType.DMA,
        ],
    )
    return f(feat, src, dst)


def kernel(feat, edge_index):
    src = edge_index[0].astype(jnp.int32)
    dst = edge_index[1].astype(jnp.int32)
    return _gather_mul(feat, src, dst)
